# Initial kernel scaffold; baseline (speedup 1.0000x reference)
import jax
import jax.numpy as jnp
from jax.experimental import pallas as pl


def kernel(x):
    raise NotImplementedError("write your pallas kernel here")


# speedup vs baseline: 81.3052x; 81.3052x over previous
import functools

import jax
import jax.numpy as jnp
from jax.experimental import pallas as pl
from jax.experimental.pallas import tpu as pltpu

_PAD = 1e-9
_W = 512
_HO = 256


def _pool_body(x_ref, s1_ref, s0_ref, s2_ref, o_ref, sc_ref):
    for k in range(4):
        sc_ref[k] = x_ref[0, :, 128 * k:128 * (k + 1)]
    xe = jnp.concatenate(
        [sc_ref[pl.ds(k, 1), pl.ds(0, _HO, 2), :][0] for k in range(4)],
        axis=1)
    xo = jnp.concatenate(
        [sc_ref[pl.ds(k, 1), pl.ds(1, _HO, 2), :][0] for k in range(4)],
        axis=1)

    ri = jax.lax.broadcasted_iota(jnp.int32, (_HO, _W), 0)
    jj = jax.lax.broadcasted_iota(jnp.int32, (_HO, _W), 1)
    xe = jnp.where(2 * ri + jj == _W, _PAD, xe)
    xo = jnp.where(2 * ri + 1 + jj == _W, _PAD, xo)

    a = jnp.concatenate([jnp.full((1, _W), _PAD, jnp.float32), xo[:-1, :]], axis=0)

    m1 = jnp.maximum(xe, xo)
    m0 = jnp.maximum(a, m1)
    m2 = jnp.maximum(a, xe)

    out = jnp.dot(m0, s0_ref[...], preferred_element_type=jnp.float32)
    out = jnp.maximum(out, jnp.dot(m1, s1_ref[...],
                                   preferred_element_type=jnp.float32))
    out = jnp.maximum(out, jnp.dot(m2, s2_ref[...],
                                   preferred_element_type=jnp.float32))

    io = jax.lax.broadcasted_iota(jnp.int32, (_HO, _HO), 0)
    jo = jax.lax.broadcasted_iota(jnp.int32, (_HO, _HO), 1)
    out = jnp.where(jo == 0, jnp.maximum(out, _PAD), out)

    o_ref[0] = jnp.where(io + jo < _HO, out, 0.0)


@jax.jit
def kernel(x):
    shape_bac = x.shape[:-2]
    n = 1
    for d in shape_bac:
        n *= d
    xf = x.reshape(n, _W, _W)

    c = jnp.arange(_W, dtype=jnp.int32)[:, None]
    oj2 = 2 * jnp.arange(_HO, dtype=jnp.int32)[None, :]
    s1 = (c == oj2 - 1).astype(jnp.float32)
    s0 = (c == oj2).astype(jnp.float32)
    s2 = (c == oj2 + 1).astype(jnp.float32)

    sel_spec = pl.BlockSpec((_W, _HO), lambda i: (0, 0))
    out = pl.pallas_call(
        _pool_body,
        grid=(n,),
        in_specs=[
            pl.BlockSpec((1, _W, _W), lambda i: (i, 0, 0)),
            sel_spec, sel_spec, sel_spec,
        ],
        out_specs=pl.BlockSpec((1, _HO, _HO), lambda i: (i, 0, 0)),
        out_shape=jax.ShapeDtypeStruct((n, _HO, _HO), jnp.float32),
        scratch_shapes=[pltpu.VMEM((4, _W, 128), jnp.float32)],
        compiler_params=pltpu.CompilerParams(
            dimension_semantics=("parallel",),
        ),
        name="hex_max_pool",
    )(xf, s1, s0, s2)

    return out.reshape(*shape_bac, _HO, _HO)


# speedup vs baseline: 142.3975x; 1.7514x over previous
import jax
import jax.numpy as jnp
from jax.experimental import pallas as pl
from jax.experimental.pallas import tpu as pltpu

_PAD = 1e-9
_W = 512
_HO = 256
_B = 4


def _pool_body(x_ref, s1_ref, s0_ref, s2_ref, o_ref, sc_ref):
    rows = _B * _HO

    for k in range(4):
        sc_ref[k] = x_ref[:, 128 * k:128 * (k + 1)]
    xe = jnp.concatenate(
        [sc_ref[pl.ds(k, 1), pl.ds(0, rows, 2), :][0] for k in range(4)],
        axis=1)
    xo = jnp.concatenate(
        [sc_ref[pl.ds(k, 1), pl.ds(1, rows, 2), :][0] for k in range(4)],
        axis=1)

    ri = jax.lax.broadcasted_iota(jnp.int32, (rows, _W), 0)
    io = ri & (_HO - 1)
    jj = jax.lax.broadcasted_iota(jnp.int32, (rows, _W), 1)
    xe = jnp.where(2 * io + jj == _W, _PAD, xe)
    xo = jnp.where(2 * io + 1 + jj == _W, _PAD, xo)

    a = jnp.concatenate(
        [jnp.full((1, _W), _PAD, jnp.float32), xo[:-1, :]], axis=0)
    a = jnp.where(io == 0, _PAD, a)

    m1 = jnp.maximum(xe, xo)
    m0 = jnp.maximum(a, m1)
    m2 = jnp.maximum(a, xe)

    out = jnp.dot(m0, s0_ref[...], preferred_element_type=jnp.float32)
    out = jnp.maximum(out, jnp.dot(m1, s1_ref[...],
                                   preferred_element_type=jnp.float32))
    out = jnp.maximum(out, jnp.dot(m2, s2_ref[...],
                                   preferred_element_type=jnp.float32))

    jo = jax.lax.broadcasted_iota(jnp.int32, (rows, _HO), 1)
    iom = jax.lax.broadcasted_iota(jnp.int32, (rows, _HO), 0) & (_HO - 1)
    out = jnp.where(jo == 0, jnp.maximum(out, _PAD), out)

    o_ref[...] = jnp.where(iom + jo < _HO, out, 0.0)


@jax.jit
def kernel(x):
    shape_bac = x.shape[:-2]
    n = 1
    for d in shape_bac:
        n *= d
    xf = x.reshape(n * _W, _W)

    c = jnp.arange(_W, dtype=jnp.int32)[:, None]
    oj2 = 2 * jnp.arange(_HO, dtype=jnp.int32)[None, :]
    s1 = (c == oj2 - 1).astype(jnp.float32)
    s0 = (c == oj2).astype(jnp.float32)
    s2 = (c == oj2 + 1).astype(jnp.float32)

    sel_spec = pl.BlockSpec((_W, _HO), lambda i: (0, 0))
    out = pl.pallas_call(
        _pool_body,
        grid=(n // _B,),
        in_specs=[
            pl.BlockSpec((_B * _W, _W), lambda i: (i, 0)),
            sel_spec, sel_spec, sel_spec,
        ],
        out_specs=pl.BlockSpec((_B * _HO, _HO), lambda i: (i, 0)),
        out_shape=jax.ShapeDtypeStruct((n * _HO, _HO), jnp.float32),
        scratch_shapes=[pltpu.VMEM((4, _B * _W, 128), jnp.float32)],
        compiler_params=pltpu.CompilerParams(
            dimension_semantics=("parallel",),
        ),
        name="hex_max_pool",
    )(xf, s1, s0, s2)

    return out.reshape(*shape_bac, _HO, _HO)


# speedup vs baseline: 163.3477x; 1.1471x over previous
import jax
import jax.numpy as jnp
from jax.experimental import pallas as pl
from jax.experimental.pallas import tpu as pltpu

_PAD = 1e-9
_W = 512
_HO = 256
_B = 8


def _pool_body(x_ref, s1_ref, s0_ref, s2_ref, o_ref, sc_ref):
    rows = _B * _HO

    for k in range(4):
        sc_ref[k] = x_ref[:, 128 * k:128 * (k + 1)]
    xe = jnp.concatenate(
        [sc_ref[pl.ds(k, 1), pl.ds(0, rows, 2), :][0] for k in range(4)],
        axis=1)
    xo = jnp.concatenate(
        [sc_ref[pl.ds(k, 1), pl.ds(1, rows, 2), :][0] for k in range(4)],
        axis=1)

    ri = jax.lax.broadcasted_iota(jnp.int32, (rows, _W), 0)
    io = ri & (_HO - 1)
    jj = jax.lax.broadcasted_iota(jnp.int32, (rows, _W), 1)
    xe = jnp.where(2 * io + jj == _W, _PAD, xe)
    xo = jnp.where(2 * io + 1 + jj == _W, _PAD, xo)

    a = jnp.concatenate(
        [jnp.full((1, _W), _PAD, jnp.float32), xo[:-1, :]], axis=0)
    a = jnp.where(io == 0, _PAD, a)

    m1 = jnp.maximum(xe, xo)
    m0 = jnp.maximum(a, m1)
    m2 = jnp.maximum(a, xe)

    out = jnp.dot(m0, s0_ref[...], preferred_element_type=jnp.float32)
    out = jnp.maximum(out, jnp.dot(m1, s1_ref[...],
                                   preferred_element_type=jnp.float32))
    out = jnp.maximum(out, jnp.dot(m2, s2_ref[...],
                                   preferred_element_type=jnp.float32))

    jo = jax.lax.broadcasted_iota(jnp.int32, (rows, _HO), 1)
    iom = jax.lax.broadcasted_iota(jnp.int32, (rows, _HO), 0) & (_HO - 1)
    out = jnp.where(jo == 0, jnp.maximum(out, _PAD), out)

    o_ref[...] = jnp.where(iom + jo < _HO, out, 0.0)


@jax.jit
def kernel(x):
    shape_bac = x.shape[:-2]
    n = 1
    for d in shape_bac:
        n *= d
    xf = x.reshape(n * _W, _W)

    c = jnp.arange(_W, dtype=jnp.int32)[:, None]
    oj2 = 2 * jnp.arange(_HO, dtype=jnp.int32)[None, :]
    s1 = (c == oj2 - 1).astype(jnp.float32)
    s0 = (c == oj2).astype(jnp.float32)
    s2 = (c == oj2 + 1).astype(jnp.float32)

    sel_spec = pl.BlockSpec((_W, _HO), lambda i: (0, 0))
    out = pl.pallas_call(
        _pool_body,
        grid=(n // _B,),
        in_specs=[
            pl.BlockSpec((_B * _W, _W), lambda i: (i, 0)),
            sel_spec, sel_spec, sel_spec,
        ],
        out_specs=pl.BlockSpec((_B * _HO, _HO), lambda i: (i, 0)),
        out_shape=jax.ShapeDtypeStruct((n * _HO, _HO), jnp.float32),
        scratch_shapes=[pltpu.VMEM((4, _B * _W, 128), jnp.float32)],
        compiler_params=pltpu.CompilerParams(
            dimension_semantics=("parallel",),
        ),
        name="hex_max_pool",
    )(xf, s1, s0, s2)

    return out.reshape(*shape_bac, _HO, _HO)
